# Initial kernel scaffold; baseline (speedup 1.0000x reference)
import jax
import jax.numpy as jnp
from jax.experimental import pallas as pl


def kernel(f_node, f_edge, edges, soe, params):
    raise NotImplementedError("write your pallas kernel here")


# speedup vs baseline: 4.6132x; 4.6132x over previous
import functools
import math

import jax
import jax.numpy as jnp
from jax.experimental import pallas as pl

D = 128
H = 4
DH = D // H

_SQRT2 = math.sqrt(2.0)


def _gelu(x):
    return 0.5 * x * (1.0 + jax.lax.erf(x / _SQRT2))


def _lnorm(x, g, b, eps=1e-5):
    m = jnp.mean(x, axis=-1, keepdims=True)
    xc = x - m
    v = jnp.mean(xc * xc, axis=-1, keepdims=True)
    return xc * jax.lax.rsqrt(v + eps) * g + b


def _rows(n_rows, bs):
    return pl.BlockSpec((bs, n_rows), lambda i: (i, 0)) if False else None


def _row_spec(bs, ncols):
    return pl.BlockSpec((bs, ncols), lambda i: (i, 0))


def _full_spec(shape):
    nd = len(shape)
    return pl.BlockSpec(shape, lambda i: (0,) * nd)


def _node_proj_body(fn_ref, ws_ref, wd_ref, wq_ref, bq_ref, a_ref, b_ref, q_ref):
    x = fn_ref[...]
    a_ref[...] = jnp.dot(x, ws_ref[...].T, preferred_element_type=jnp.float32)
    b_ref[...] = jnp.dot(x, wd_ref[...].T, preferred_element_type=jnp.float32)
    q_ref[...] = jnp.dot(x, wq_ref[...].T, preferred_element_type=jnp.float32) + bq_ref[...]


def _edge_mid_body(fe_ref, wm_ref, bc_ref, out_ref):
    out_ref[...] = (
        jnp.dot(fe_ref[...], wm_ref[...].T, preferred_element_type=jnp.float32)
        + bc_ref[...]
    )


def _fec_u1_body(ga_ref, gb_ref, mid_ref, eg_ref, eb_ref, wcu_ref, bcu_ref,
                 cg_ref, cb_ref, fec_ref, u1_ref):
    s = ga_ref[...] + mid_ref[...] + gb_ref[...]
    fec = _lnorm(s, eg_ref[...], eb_ref[...])
    fec_ref[...] = fec
    t = jnp.dot(fec, wcu_ref[...].T, preferred_element_type=jnp.float32) + bcu_ref[...]
    u1_ref[...] = _lnorm(_gelu(t), cg_ref[...], cb_ref[...])


def _tri_body(g0_ref, g1_ref, g2_ref, w0_ref, w1_ref, w2_ref, bt_ref, out_ref):
    acc = (
        jnp.dot(g0_ref[...], w0_ref[...].T, preferred_element_type=jnp.float32)
        + jnp.dot(g1_ref[...], w1_ref[...].T, preferred_element_type=jnp.float32)
        + jnp.dot(g2_ref[...], w2_ref[...].T, preferred_element_type=jnp.float32)
        + bt_ref[...]
    )
    out_ref[...] = _gelu(acc)


def _mega_body(fec_ref, fsh_ref, u1_ref, tsum_ref, tcnt_ref, fe_ref, qg_ref,
               ws1_ref, ws2_ref, bsym_ref, sg_ref, sb_ref,
               tg_ref, tb_ref,
               fg_ref, fb_ref, wsq_ref, bsq_ref, wex_ref, bex_ref,
               ug_ref, ub_ref, w1_ref, w2_ref, ng_ref, nb_ref,
               wk_ref, bk_ref, wv_ref, bv_ref,
               fe2_ref, v_ref, logit_ref):
    fec = fec_ref[...]
    u1 = u1_ref[...]
    t = (
        jnp.dot(fec, ws1_ref[...].T, preferred_element_type=jnp.float32)
        + jnp.dot(fsh_ref[...], ws2_ref[...].T, preferred_element_type=jnp.float32)
        + bsym_ref[...]
    )
    u2 = _lnorm(_gelu(t), sg_ref[...], sb_ref[...])
    cnt = tcnt_ref[...]
    u3 = _lnorm(tsum_ref[...] / jnp.maximum(cnt, 1.0), tg_ref[...], tb_ref[...])
    s = u1 + u2 + u3
    z = _lnorm(s, fg_ref[...], fb_ref[...])
    z = jnp.dot(z, wsq_ref[...].T, preferred_element_type=jnp.float32) + bsq_ref[...]
    z = jnp.dot(_gelu(z), wex_ref[...].T, preferred_element_type=jnp.float32) + bex_ref[...]
    a0 = z[:, :D]
    a1 = z[:, D:2 * D]
    a2 = z[:, 2 * D:]
    m = jnp.maximum(jnp.maximum(a0, a1), a2)
    e0 = jnp.exp(a0 - m)
    e1 = jnp.exp(a1 - m)
    e2 = jnp.exp(a2 - m)
    inv = 1.0 / (e0 + e1 + e2)
    f_up = (u1 * e0 + u2 * e1 + u3 * e2) * inv
    fe = _lnorm(fe_ref[...] + f_up, ug_ref[...], ub_ref[...])
    y = jnp.dot(fe, w1_ref[...].T, preferred_element_type=jnp.float32)
    h = _gelu(y[:, 2 * D:]) * y[:, :2 * D]
    fe2 = _lnorm(
        fe + jnp.dot(h, w2_ref[...].T, preferred_element_type=jnp.float32),
        ng_ref[...], nb_ref[...])
    fe2_ref[...] = fe2
    k = jnp.dot(fe2, wk_ref[...].T, preferred_element_type=jnp.float32) + bk_ref[...]
    v_ref[...] = jnp.dot(fe2, wv_ref[...].T, preferred_element_type=jnp.float32) + bv_ref[...]
    qk = qg_ref[...] * k
    scale = DH ** -0.5
    qk3 = qk.reshape(qk.shape[0], H, DH)
    logit_ref[...] = jnp.sum(qk3, axis=-1) * scale


def _node_out_body(fn_ref, agg_ref, wo_ref, bo_ref, ag_ref, ab_ref,
                   w1_ref, w2_ref, ng_ref, nb_ref, out_ref):
    fn = fn_ref[...]
    t = jnp.dot(agg_ref[...], wo_ref[...].T, preferred_element_type=jnp.float32) + bo_ref[...]
    x = _lnorm(fn + t, ag_ref[...], ab_ref[...])
    y = jnp.dot(x, w1_ref[...].T, preferred_element_type=jnp.float32)
    h = _gelu(y[:, 2 * D:]) * y[:, :2 * D]
    out_ref[...] = _lnorm(
        x + jnp.dot(h, w2_ref[...].T, preferred_element_type=jnp.float32),
        ng_ref[...], nb_ref[...])


def _r1(x):
    return x.reshape(1, -1)


def kernel(f_node, f_edge, edges, soe, params):
    p = params
    L = f_node.shape[0]
    E = f_edge.shape[0]
    T = soe.shape[0]
    src = edges[0]
    dst = edges[1]

    BL = 2000
    BE = 2000
    GE = E // BE

    Wc = p['eu_connection']['W']
    bc = p['eu_connection']['b']
    Wq = p['nu_q']['W']
    bq = p['nu_q']['b']

    a_n, b_n, q_n = pl.pallas_call(
        _node_proj_body,
        grid=(L // BL,),
        in_specs=[
            _row_spec(BL, D),
            _full_spec((D, D)), _full_spec((D, D)), _full_spec((D, D)),
            _full_spec((1, D)),
        ],
        out_specs=[_row_spec(BL, D)] * 3,
        out_shape=[jax.ShapeDtypeStruct((L, D), jnp.float32)] * 3,
    )(f_node, Wc[:, :D], Wc[:, 2 * D:], Wq, _r1(bq))

    mid = pl.pallas_call(
        _edge_mid_body,
        grid=(GE,),
        in_specs=[_row_spec(BE, D), _full_spec((D, D)), _full_spec((1, D))],
        out_specs=_row_spec(BE, D),
        out_shape=jax.ShapeDtypeStruct((E, D), jnp.float32),
    )(f_edge, Wc[:, D:2 * D], _r1(bc))

    ga = jnp.take(a_n, src, axis=0)
    gb = jnp.take(b_n, dst, axis=0)
    qg = jnp.take(q_n, dst, axis=0)

    fec, u1 = pl.pallas_call(
        _fec_u1_body,
        grid=(GE,),
        in_specs=[
            _row_spec(BE, D), _row_spec(BE, D), _row_spec(BE, D),
            _full_spec((1, D)), _full_spec((1, D)),
            _full_spec((D, D)), _full_spec((1, D)),
            _full_spec((1, D)), _full_spec((1, D)),
        ],
        out_specs=[_row_spec(BE, D)] * 2,
        out_shape=[jax.ShapeDtypeStruct((E, D), jnp.float32)] * 2,
    )(ga, gb, mid,
      _r1(p['eu_norm']['g']), _r1(p['eu_norm']['b']),
      p['eu_cu_lin']['W'], _r1(p['eu_cu_lin']['b']),
      _r1(p['eu_cu_norm']['g']), _r1(p['eu_cu_norm']['b']))

    g0 = jnp.take(fec, soe[:, 0], axis=0)
    g1 = jnp.take(fec, soe[:, 1], axis=0)
    g2 = jnp.take(fec, soe[:, 2], axis=0)
    Wt = p['eu_tri_lin']['W']
    BT = 2000
    f_tri = pl.pallas_call(
        _tri_body,
        grid=(T // BT,),
        in_specs=[
            _row_spec(BT, D), _row_spec(BT, D), _row_spec(BT, D),
            _full_spec((D, D)), _full_spec((D, D)), _full_spec((D, D)),
            _full_spec((1, D)),
        ],
        out_specs=_row_spec(BT, D),
        out_shape=jax.ShapeDtypeStruct((T, D), jnp.float32),
    )(g0, g1, g2, Wt[:, :D], Wt[:, D:2 * D], Wt[:, 2 * D:], _r1(p['eu_tri_lin']['b']))

    tid = soe[:, 2]
    tsum = jax.ops.segment_sum(f_tri, tid, E)
    tcnt = jax.ops.segment_sum(jnp.ones((T, 1), jnp.float32), tid, E)

    Wsym = p['eu_sym_lin']['W']
    half = GE // 2
    fsh_spec = pl.BlockSpec((BE, D), lambda i: ((i + half) % GE, 0))
    fe2, v, logits = pl.pallas_call(
        _mega_body,
        grid=(GE,),
        in_specs=[
            _row_spec(BE, D), fsh_spec, _row_spec(BE, D), _row_spec(BE, D),
            pl.BlockSpec((BE, 1), lambda i: (i, 0)),
            _row_spec(BE, D), _row_spec(BE, D),
            _full_spec((D, D)), _full_spec((D, D)), _full_spec((1, D)),
            _full_spec((1, D)), _full_spec((1, D)),
            _full_spec((1, D)), _full_spec((1, D)),
            _full_spec((1, D)), _full_spec((1, D)),
            _full_spec((D // 4, D)), _full_spec((1, D // 4)),
            _full_spec((3 * D, D // 4)), _full_spec((1, 3 * D)),
            _full_spec((1, D)), _full_spec((1, D)),
            _full_spec((4 * D, D)), _full_spec((D, 2 * D)),
            _full_spec((1, D)), _full_spec((1, D)),
            _full_spec((D, D)), _full_spec((1, D)),
            _full_spec((D, D)), _full_spec((1, D)),
        ],
        out_specs=[_row_spec(BE, D), _row_spec(BE, D), _row_spec(BE, H)],
        out_shape=[
            jax.ShapeDtypeStruct((E, D), jnp.float32),
            jax.ShapeDtypeStruct((E, D), jnp.float32),
            jax.ShapeDtypeStruct((E, H), jnp.float32),
        ],
    )(fec, fec, u1, tsum, tcnt, f_edge, qg,
      Wsym[:, :D], Wsym[:, D:], _r1(p['eu_sym_lin']['b']),
      _r1(p['eu_sym_norm']['g']), _r1(p['eu_sym_norm']['b']),
      _r1(p['eu_tri_norm']['g']), _r1(p['eu_tri_norm']['b']),
      _r1(p['sk_norm_fea']['g']), _r1(p['sk_norm_fea']['b']),
      p['sk_squeeze']['W'], _r1(p['sk_squeeze']['b']),
      p['sk_excitation']['W'], _r1(p['sk_excitation']['b']),
      _r1(p['sk_norm_update']['g']), _r1(p['sk_norm_update']['b']),
      p['eu_ff1']['W'], p['eu_ff2']['W'],
      _r1(p['eu_ff_norm']['g']), _r1(p['eu_ff_norm']['b']),
      p['nu_k']['W'], _r1(p['nu_k']['b']),
      p['nu_v']['W'], _r1(p['nu_v']['b']))

    mx = jax.ops.segment_max(logits, dst, L)
    mx = jnp.where(jnp.isneginf(mx), 0.0, mx)
    e = jnp.exp(logits - mx[dst])
    ssum = jax.ops.segment_sum(e, dst, L)
    attn = e / (ssum[dst] + 1e-7)
    w = jnp.repeat(attn, DH, axis=1)
    f_agg = jax.ops.segment_sum(w * v, dst, L)

    fn = pl.pallas_call(
        _node_out_body,
        grid=(L // BL,),
        in_specs=[
            _row_spec(BL, D), _row_spec(BL, D),
            _full_spec((D, D)), _full_spec((1, D)),
            _full_spec((1, D)), _full_spec((1, D)),
            _full_spec((4 * D, D)), _full_spec((D, 2 * D)),
            _full_spec((1, D)), _full_spec((1, D)),
        ],
        out_specs=_row_spec(BL, D),
        out_shape=jax.ShapeDtypeStruct((L, D), jnp.float32),
    )(f_node, f_agg,
      p['nu_out']['W'], _r1(p['nu_out']['b']),
      _r1(p['nu_attn_norm']['g']), _r1(p['nu_attn_norm']['b']),
      p['nu_ff1']['W'], p['nu_ff2']['W'],
      _r1(p['nu_ff_norm']['g']), _r1(p['nu_ff_norm']['b']))

    return fn, fe2
